# Initial kernel scaffold; baseline (speedup 1.0000x reference)
import jax
import jax.numpy as jnp
from jax.experimental import pallas as pl


def kernel(embeddings1_c, embeddings1_s, embeddings2_c, embeddings2_s, gt_corr_ms, numPlanes1, numPlanes2, loss_weight):
    raise NotImplementedError("write your pallas kernel here")


# speedup vs baseline: 3.3045x; 3.3045x over previous
import functools
import jax
import jax.numpy as jnp
from jax import lax
from jax.experimental import pallas as pl
from jax.experimental.pallas import tpu as pltpu

MARGIN_C = 0.2
BB = 8


def _tc_body(e1c, e1s, e2c, e2s, g, n1, n2, lw, out, acc):
    gi = pl.program_id(0)
    ng = pl.num_programs(0)

    @pl.when(gi == 0)
    def _init():
        acc[0, 0] = 0.0
        acc[0, 1] = 0.0

    total = jnp.zeros((1, 1), jnp.float32)
    cnt = jnp.zeros((1, 1), jnp.float32)
    for k in range(BB):
        a_c = e1c[k]
        a_s = e1s[k]
        b_c = e2c[k]
        b_s = e2s[k]

        ones_row = jnp.ones((1, a_c.shape[1]), jnp.float32)

        def pdist(a, b):
            q1 = jnp.sum(a * a, axis=1, keepdims=True)
            q2 = lax.dot_general(ones_row, b * b, (((1,), (1,)), ((), ())),
                                 preferred_element_type=jnp.float32)
            dots = lax.dot_general(a, b, (((1,), (1,)), ((), ())),
                                   preferred_element_type=jnp.float32)
            d2 = q1 + q2 - 2.0 * dots
            return jnp.sqrt(jnp.maximum(d2, 1e-12))

        dc = pdist(a_c, b_c)
        ds = pdist(a_s, b_s)
        w = 1.0 - 2.0 * jnp.minimum(ds * 0.5, 1.0) ** 2
        dist = (1.0 - w) * dc + w * ds

        p1 = dist.shape[0]
        p2 = dist.shape[1]
        row_ok = lax.broadcasted_iota(jnp.int32, (p1, p2), 0) < n1[gi * BB + k, 0]
        col_ok = lax.broadcasted_iota(jnp.int32, (p1, p2), 1) < n2[gi * BB + k, 0]
        dm = jnp.where(row_ok & col_ok, dist, 100.0)

        gk = g[k] > 0.0
        minmask = jnp.where(gk, 1e30, dm)
        m = jnp.min(minmask, axis=1, keepdims=True)
        t = dm - m + MARGIN_C
        contrib = jnp.where(gk, jnp.maximum(t, 0.0), 0.0)
        kept = jnp.where(gk & (t > 0.0), 1.0, 0.0)
        total = total + jnp.sum(contrib, keepdims=True).reshape(1, 1)
        cnt = cnt + jnp.sum(kept, keepdims=True).reshape(1, 1)

    acc[0, 0] += total[0, 0]
    acc[0, 1] += cnt[0, 0]

    @pl.when(gi == ng - 1)
    def _fin():
        tot = acc[0, 0]
        c = acc[0, 1]
        mean = jnp.where(c > 0.0, tot / jnp.maximum(c, 1.0), MARGIN_C)
        out[0, 0] = lw[0] * mean


@jax.jit
def _run(e1c, e1s, e2c, e2s, gf, n1, n2, lw):
    b = e1c.shape[0]
    grid = (b // BB,)
    emb_spec = pl.BlockSpec((BB, e1c.shape[1], e1c.shape[2]), lambda i: (i, 0, 0))
    g_spec = pl.BlockSpec((BB, gf.shape[1], gf.shape[2]), lambda i: (i, 0, 0))
    smem = functools.partial(pl.BlockSpec, memory_space=pltpu.SMEM)
    out = pl.pallas_call(
        _tc_body,
        grid=grid,
        in_specs=[emb_spec, emb_spec, emb_spec, emb_spec, g_spec,
                  smem(), smem(), smem()],
        out_specs=pl.BlockSpec((1, 2), lambda i: (0, 0), memory_space=pltpu.SMEM),
        out_shape=jax.ShapeDtypeStruct((1, 2), jnp.float32),
        scratch_shapes=[pltpu.SMEM((1, 2), jnp.float32)],
    )(e1c, e1s, e2c, e2s, gf, n1, n2, lw)
    return out[0, 0]


def kernel(embeddings1_c, embeddings1_s, embeddings2_c, embeddings2_s,
           gt_corr_ms, numPlanes1, numPlanes2, loss_weight):
    gf = gt_corr_ms.astype(jnp.float32)
    lw = jnp.asarray(loss_weight, jnp.float32).reshape(1)
    return _run(embeddings1_c, embeddings1_s, embeddings2_c, embeddings2_s,
                gf, numPlanes1, numPlanes2, lw)
